# baseline (speedup 1.0000x reference)
import functools

import jax
import jax.numpy as jnp
from jax import lax
from jax.experimental import pallas as pl
from jax.experimental.pallas import tpu as pltpu
from jax.experimental.pallas import tpu_sc as plsc

_mesh = plsc.ScalarSubcoreMesh(axis_name="c", num_cores=2)


@functools.partial(
    pl.kernel,
    mesh=_mesh,
    out_type=jax.ShapeDtypeStruct((3, 128), jnp.float32),
    scratch_types=[pltpu.SemaphoreType.DMA],
)
def _gather_rows(x_hbm, out_hbm, sem):
    cid = lax.axis_index("c")

    @pl.when(cid == 0)
    def _():
        pltpu.sync_copy(x_hbm.at[pl.ds(4, 2)], out_hbm.at[pl.ds(1, 2)])

    @pl.when(cid == 1)
    def _():
        pltpu.sync_copy(x_hbm.at[pl.ds(2, 1)], out_hbm.at[pl.ds(0, 1)])


def kernel(x):
    return _gather_rows(x)


# speedup vs baseline: 1.0873x; 1.0873x over previous
import functools

import jax
import jax.numpy as jnp
from jax import lax
from jax.experimental import pallas as pl
from jax.experimental.pallas import tpu as pltpu
from jax.experimental.pallas import tpu_sc as plsc

_mesh = plsc.ScalarSubcoreMesh(axis_name="c", num_cores=1)


@functools.partial(
    pl.kernel,
    mesh=_mesh,
    out_type=jax.ShapeDtypeStruct((3, 128), jnp.float32),
    scratch_types=[pltpu.SemaphoreType.DMA],
)
def _gather_rows(x_hbm, out_hbm, sem):
    c2 = pltpu.async_copy(x_hbm.at[pl.ds(4, 2)], out_hbm.at[pl.ds(1, 2)], sem)
    c1 = pltpu.async_copy(x_hbm.at[pl.ds(2, 1)], out_hbm.at[pl.ds(0, 1)], sem)
    c2.wait()
    c1.wait()


def kernel(x):
    return _gather_rows(x)


# speedup vs baseline: 1.0882x; 1.0008x over previous
import functools

import jax
import jax.numpy as jnp
from jax.experimental import pallas as pl
from jax.experimental.pallas import tpu as pltpu
from jax.experimental.pallas import tpu_sc as plsc

_mesh = plsc.ScalarSubcoreMesh(axis_name="c", num_cores=1)


@functools.partial(
    pl.kernel,
    mesh=_mesh,
    out_type=jax.ShapeDtypeStruct((3, 128), jnp.float32),
    scratch_types=[pltpu.SemaphoreType.DMA],
)
def _gather_rows(x_hbm, out_hbm, sem):
    c2 = pltpu.async_copy(x_hbm.at[pl.ds(4, 2)], out_hbm.at[pl.ds(1, 2)], sem)
    c1 = pltpu.async_copy(x_hbm.at[pl.ds(2, 1)], out_hbm.at[pl.ds(0, 1)], sem)
    c2.wait()
    c1.wait()


def kernel(x):
    return _gather_rows(x)
